# Initial kernel scaffold; baseline (speedup 1.0000x reference)
import functools

import jax
import jax.numpy as jnp
from jax import lax
from jax.experimental import pallas as pl
from jax.experimental.pallas import tpu as pltpu
from jax.experimental.pallas import tpu_sc as plsc

N = 10000
D = 128
E = 320000
NC = 2
NS = 16
LANES = 16
CHUNK = 80
EDGES_PER_TILE = E // (NC * NS)
NCHUNK = EDGES_PER_TILE // CHUNK
ROWS_PER_TILE = N // NS


def _linear_body(x_ref, w_ref, b_ref, o_ref):
    o_ref[...] = (
        lax.dot_general(x_ref[...], w_ref[...], (((1,), (1,)), ((), ())),
                        preferred_element_type=jnp.float32)
        + b_ref[...]
    )


def _linear(X0, W, b2):
    return pl.pallas_call(
        _linear_body,
        grid=(10,),
        in_specs=[
            pl.BlockSpec((1000, D), lambda i: (i, 0)),
            pl.BlockSpec((D, D), lambda i: (0, 0)),
            pl.BlockSpec((1, D), lambda i: (0, 0)),
        ],
        out_specs=pl.BlockSpec((1000, D), lambda i: (i, 0)),
        out_shape=jax.ShapeDtypeStruct((N, D), jnp.float32),
    )(X0, W, b2)


_MESH = plsc.VectorSubcoreMesh(core_axis_name="c", subcore_axis_name="s")


@functools.partial(
    pl.kernel,
    out_type=jax.ShapeDtypeStruct((NC, N, D), jnp.float32),
    mesh=_MESH,
    scratch_types=[
        pltpu.VMEM((CHUNK,), jnp.int32),
        pltpu.VMEM((CHUNK,), jnp.int32),
        pltpu.VMEM((CHUNK,), jnp.float32),
        pltpu.VMEM((CHUNK, D), jnp.float32),
        pltpu.VMEM_SHARED((N, D), jnp.float32),
        pltpu.SemaphoreType.DMA,
    ],
)
def _spmm(h_hbm, col_hbm, row_hbm, w_hbm, zeros_hbm, out_hbm,
          colv, rowv, wv, rows, acc, sem):
    c = lax.axis_index("c")
    s = lax.axis_index("s")

    pltpu.sync_copy(zeros_hbm.at[pl.ds(s * ROWS_PER_TILE, ROWS_PER_TILE)],
                    acc.at[pl.ds(s * ROWS_PER_TILE, ROWS_PER_TILE)])
    plsc.subcore_barrier()

    base = (c * NS + s) * EDGES_PER_TILE

    def chunk_body(i, carry):
        e0 = base + i * CHUNK
        pltpu.sync_copy(col_hbm.at[pl.ds(e0, CHUNK)], colv)
        pltpu.sync_copy(row_hbm.at[pl.ds(e0, CHUNK)], rowv)
        pltpu.sync_copy(w_hbm.at[pl.ds(e0, CHUNK)], wv)
        pltpu.async_copy(h_hbm.at[colv], rows, sem).wait()

        def edge_body(e, carry2):
            wb = plsc.load_gather(wv, [jnp.full((LANES,), e, jnp.int32)])
            for j in range(D // LANES):
                sl = rows[e, pl.ds(j * LANES, LANES)]
                rows[e, pl.ds(j * LANES, LANES)] = sl * wb
            return carry2

        lax.fori_loop(0, CHUNK, edge_body, 0, unroll=2)

        pltpu.sync_copy(rows, acc.at[rowv], add=True)
        return carry

    lax.fori_loop(0, NCHUNK, chunk_body, 0)

    plsc.subcore_barrier()
    pltpu.sync_copy(acc.at[pl.ds(s * ROWS_PER_TILE, ROWS_PER_TILE)],
                    out_hbm.at[c, pl.ds(s * ROWS_PER_TILE, ROWS_PER_TILE)])


def _combine_body(p_ref, o_ref):
    o_ref[...] = jnp.maximum(p_ref[0] + p_ref[1], 0.0)


def _combine(partials):
    return pl.pallas_call(
        _combine_body,
        grid=(10,),
        in_specs=[pl.BlockSpec((NC, 1000, D), lambda i: (0, i, 0))],
        out_specs=pl.BlockSpec((1000, D), lambda i: (i, 0)),
        out_shape=jax.ShapeDtypeStruct((N, D), jnp.float32),
    )(partials)


def kernel(X0, edge_index, edge_weight, W, b):
    ei = edge_index.astype(jnp.int32)
    row = ei[0]
    col = ei[1]
    h = _linear(X0, W, b.reshape(1, D))
    zeros = jnp.zeros((N, D), jnp.float32)
    partials = _spmm(h, col, row, edge_weight, zeros)
    return _combine(partials)


# speedup vs baseline: 7.6048x; 7.6048x over previous
import functools

import jax
import jax.numpy as jnp
from jax import lax
from jax.experimental import pallas as pl
from jax.experimental.pallas import tpu as pltpu
from jax.experimental.pallas import tpu_sc as plsc

N = 10000
D = 128
E = 320000
NC = 2
NS = 16
NW = NC * NS
LANES = 16
CHUNK = 80
NROWS_E = E // CHUNK
CPB = 8
RSTRIDE = NW * CPB
FULL_ROUNDS = NROWS_E // RSTRIDE
EXTRA_TILES = (NROWS_E - FULL_ROUNDS * RSTRIDE) // CPB
ROWS_PER_TILE = 624
ROWS_TAIL = N - NS * ROWS_PER_TILE


def _linear_body(x_ref, w_ref, b_ref, o_ref):
    o_ref[...] = (
        lax.dot_general(x_ref[...], w_ref[...], (((1,), (1,)), ((), ())),
                        preferred_element_type=jnp.float32)
        + b_ref[...]
    )


def _linear(X0, W, b2):
    return pl.pallas_call(
        _linear_body,
        grid=(10,),
        in_specs=[
            pl.BlockSpec((1000, D), lambda i: (i, 0)),
            pl.BlockSpec((D, D), lambda i: (0, 0)),
            pl.BlockSpec((1, D), lambda i: (0, 0)),
        ],
        out_specs=pl.BlockSpec((1000, D), lambda i: (i, 0)),
        out_shape=jax.ShapeDtypeStruct((N, D), jnp.float32),
    )(X0, W, b2)


_MESH = plsc.VectorSubcoreMesh(core_axis_name="c", subcore_axis_name="s")


@functools.partial(
    pl.kernel,
    out_type=jax.ShapeDtypeStruct((NC, N, D), jnp.float32),
    mesh=_MESH,
    scratch_types=[
        pltpu.VMEM((CPB, CHUNK), jnp.int32),
        pltpu.VMEM((CPB, CHUNK), jnp.int32),
        pltpu.VMEM((CPB, CHUNK), jnp.float32),
        pltpu.VMEM((CHUNK, D), jnp.float32),
        pltpu.VMEM((CHUNK, D), jnp.float32),
        pltpu.VMEM_SHARED((N, D), jnp.float32),
        pltpu.SemaphoreType.DMA,
        pltpu.SemaphoreType.DMA,
    ],
)
def _spmm(h_hbm, col_hbm, row_hbm, w_hbm, zeros_hbm, out_hbm,
          colb, rowb, wtb, rows0, rows1, acc, sem0, sem1):
    c = lax.axis_index("c")
    s = lax.axis_index("s")
    wid = c * NS + s

    pltpu.sync_copy(zeros_hbm.at[pl.ds(s * ROWS_PER_TILE, ROWS_PER_TILE)],
                    acc.at[pl.ds(s * ROWS_PER_TILE, ROWS_PER_TILE)])

    @pl.when(s == 0)
    def _init_tail():
        pltpu.sync_copy(zeros_hbm.at[pl.ds(NS * ROWS_PER_TILE, ROWS_TAIL)],
                        acc.at[pl.ds(NS * ROWS_PER_TILE, ROWS_TAIL)])

    plsc.subcore_barrier()

    bufs = (rows0, rows1)
    sems = (sem0, sem1)

    def block_body(r, carry):
        blockrow = r * RSTRIDE + wid * CPB
        pltpu.sync_copy(col_hbm.at[pl.ds(blockrow, CPB)], colb)
        pltpu.sync_copy(row_hbm.at[pl.ds(blockrow, CPB)], rowb)
        pltpu.sync_copy(w_hbm.at[pl.ds(blockrow, CPB)], wtb)

        pltpu.async_copy(h_hbm.at[colb.at[0]], bufs[0], sems[0])
        for k in range(CPB):
            buf = bufs[k % 2]
            if k + 1 < CPB:
                pltpu.async_copy(h_hbm.at[colb.at[k + 1]],
                                 bufs[(k + 1) % 2], sems[(k + 1) % 2])
            pltpu.make_async_copy(h_hbm.at[colb.at[k]], buf, sems[k % 2]).wait()

            for g in range(CHUNK // LANES):
                w16 = wtb[k, pl.ds(g * LANES, LANES)]

                def edge_body(rr, carry2, g=g, w16=w16, buf=buf):
                    wbc = w16[jnp.full((LANES,), rr, jnp.int32)]
                    e = g * LANES + rr
                    for j in range(D // LANES):
                        sl = buf[e, pl.ds(j * LANES, LANES)]
                        buf[e, pl.ds(j * LANES, LANES)] = sl * wbc
                    return carry2

                lax.fori_loop(0, LANES, edge_body, 0, unroll=2)

            pltpu.sync_copy(buf, acc.at[rowb.at[k]], add=True)
        return carry

    nblk = FULL_ROUNDS + jnp.where(wid < EXTRA_TILES, 1, 0)
    lax.fori_loop(0, nblk, block_body, 0)

    plsc.subcore_barrier()
    pltpu.sync_copy(acc.at[pl.ds(s * ROWS_PER_TILE, ROWS_PER_TILE)],
                    out_hbm.at[c, pl.ds(s * ROWS_PER_TILE, ROWS_PER_TILE)])

    @pl.when(s == 0)
    def _dump_tail():
        pltpu.sync_copy(acc.at[pl.ds(NS * ROWS_PER_TILE, ROWS_TAIL)],
                        out_hbm.at[c, pl.ds(NS * ROWS_PER_TILE, ROWS_TAIL)])


def _combine_body(p_ref, o_ref):
    o_ref[...] = jnp.maximum(p_ref[0] + p_ref[1], 0.0)


def _combine(partials):
    return pl.pallas_call(
        _combine_body,
        grid=(10,),
        in_specs=[pl.BlockSpec((NC, 1000, D), lambda i: (0, i, 0))],
        out_specs=pl.BlockSpec((1000, D), lambda i: (i, 0)),
        out_shape=jax.ShapeDtypeStruct((N, D), jnp.float32),
    )(partials)


def kernel(X0, edge_index, edge_weight, W, b):
    ei = edge_index.astype(jnp.int32)
    row2 = ei[0].reshape(NROWS_E, CHUNK)
    col2 = ei[1].reshape(NROWS_E, CHUNK)
    w2 = edge_weight.reshape(NROWS_E, CHUNK)
    h = _linear(X0, W, b.reshape(1, D))
    zeros = jnp.zeros((N, D), jnp.float32)
    partials = _spmm(h, col2, row2, w2, zeros)
    return _combine(partials)


# speedup vs baseline: 9.7949x; 1.2880x over previous
import functools

import jax
import jax.numpy as jnp
from jax import lax
from jax.experimental import pallas as pl
from jax.experimental.pallas import tpu as pltpu
from jax.experimental.pallas import tpu_sc as plsc

N = 10000
D = 128
E = 320000
NC = 2
NS = 16
NW = NC * NS
LANES = 16
CHUNK = 80
NROWS_E = E // CHUNK
CPB = 8
RSTRIDE = NW * CPB
FULL_ROUNDS = NROWS_E // RSTRIDE
EXTRA_TILES = (NROWS_E - FULL_ROUNDS * RSTRIDE) // CPB
NBUF = 4
ROWS_PER_TILE = 624
ROWS_TAIL = N - NS * ROWS_PER_TILE


def _linear_body(x_ref, w_ref, b_ref, o_ref):
    o_ref[...] = (
        lax.dot_general(x_ref[...], w_ref[...], (((1,), (1,)), ((), ())),
                        preferred_element_type=jnp.float32)
        + b_ref[...]
    )


def _linear(X0, W, b2):
    return pl.pallas_call(
        _linear_body,
        grid=(10,),
        in_specs=[
            pl.BlockSpec((1000, D), lambda i: (i, 0)),
            pl.BlockSpec((D, D), lambda i: (0, 0)),
            pl.BlockSpec((1, D), lambda i: (0, 0)),
        ],
        out_specs=pl.BlockSpec((1000, D), lambda i: (i, 0)),
        out_shape=jax.ShapeDtypeStruct((N, D), jnp.float32),
    )(X0, W, b2)


_MESH = plsc.VectorSubcoreMesh(core_axis_name="c", subcore_axis_name="s")


@functools.partial(
    pl.kernel,
    out_type=jax.ShapeDtypeStruct((NC, N, D), jnp.float32),
    mesh=_MESH,
    scratch_types=(
        [
            pltpu.VMEM((2, CPB, CHUNK), jnp.int32),
            pltpu.VMEM((2, CPB, CHUNK), jnp.int32),
            pltpu.VMEM((2, CPB, CHUNK), jnp.float32),
        ]
        + [pltpu.VMEM((CHUNK, D), jnp.float32) for _ in range(NBUF)]
        + [pltpu.VMEM_SHARED((N, D), jnp.float32)]
        + [pltpu.SemaphoreType.DMA for _ in range(2 * NBUF + 1)]
    ),
)
def _spmm(h_hbm, col_hbm, row_hbm, w_hbm, zeros_hbm, out_hbm, *scr):
    colb, rowb, wtb = scr[0], scr[1], scr[2]
    bufs = scr[3:3 + NBUF]
    acc = scr[3 + NBUF]
    gsem = scr[4 + NBUF:4 + 2 * NBUF]
    ssem = scr[4 + 2 * NBUF:4 + 3 * NBUF]
    isem = scr[4 + 3 * NBUF]

    c = lax.axis_index("c")
    s = lax.axis_index("s")
    wid = c * NS + s

    pltpu.sync_copy(zeros_hbm.at[pl.ds(s * ROWS_PER_TILE, ROWS_PER_TILE)],
                    acc.at[pl.ds(s * ROWS_PER_TILE, ROWS_PER_TILE)])

    @pl.when(s == 0)
    def _init_tail():
        pltpu.sync_copy(zeros_hbm.at[pl.ds(NS * ROWS_PER_TILE, ROWS_TAIL)],
                        acc.at[pl.ds(NS * ROWS_PER_TILE, ROWS_TAIL)])

    plsc.subcore_barrier()

    br0 = wid * CPB
    pltpu.sync_copy(col_hbm.at[pl.ds(br0, CPB)], colb.at[0])
    pltpu.sync_copy(row_hbm.at[pl.ds(br0, CPB)], rowb.at[0])
    pltpu.sync_copy(w_hbm.at[pl.ds(br0, CPB)], wtb.at[0])

    nblk = FULL_ROUNDS + jnp.where(wid < EXTRA_TILES, 1, 0)

    def block_body(r, carry):
        slot = lax.rem(r, 2)
        nslot = 1 - slot
        nbr = (r + 1) * RSTRIDE + wid * CPB

        @pl.when(r + 1 < nblk)
        def _prefetch():
            pltpu.async_copy(col_hbm.at[pl.ds(nbr, CPB)], colb.at[nslot], isem)
            pltpu.async_copy(row_hbm.at[pl.ds(nbr, CPB)], rowb.at[nslot], isem)
            pltpu.async_copy(w_hbm.at[pl.ds(nbr, CPB)], wtb.at[nslot], isem)

        def _reuse_gather(p, first_block_cond):
            b = p % NBUF

            @pl.when(first_block_cond)
            def _drain():
                pltpu.make_async_copy(bufs[b], acc.at[rowb.at[slot, p]],
                                      ssem[b]).wait()

            pltpu.async_copy(h_hbm.at[colb.at[slot, p]], bufs[b], gsem[b])

        _reuse_gather(0, r > 0)
        _reuse_gather(1, r > 0)

        for k in range(CPB):
            b = k % NBUF
            p = k + 2
            if p < CPB:
                _reuse_gather(p, (r > 0) if p < NBUF else (r >= 0))
            pltpu.make_async_copy(h_hbm.at[colb.at[slot, k]], bufs[b],
                                  gsem[b]).wait()
            for g in range(CHUNK // LANES):
                w16 = wtb[slot, k, pl.ds(g * LANES, LANES)]

                def edge_body(rr, carry2, g=g, w16=w16, b=b):
                    wbc = w16[jnp.full((LANES,), rr, jnp.int32)]
                    e = g * LANES + rr
                    for j in range(D // LANES):
                        sl = bufs[b][e, pl.ds(j * LANES, LANES)]
                        bufs[b][e, pl.ds(j * LANES, LANES)] = sl * wbc
                    return carry2

                lax.fori_loop(0, LANES, edge_body, 0, unroll=2)

            pltpu.async_copy(bufs[b], acc.at[rowb.at[slot, k]], ssem[b],
                             add=True)

        @pl.when(r + 1 < nblk)
        def _wait_prefetch():
            pltpu.make_async_copy(col_hbm.at[pl.ds(nbr, CPB)], colb.at[nslot],
                                  isem).wait()
            pltpu.make_async_copy(row_hbm.at[pl.ds(nbr, CPB)], rowb.at[nslot],
                                  isem).wait()
            pltpu.make_async_copy(w_hbm.at[pl.ds(nbr, CPB)], wtb.at[nslot],
                                  isem).wait()
        return carry

    lax.fori_loop(0, nblk, block_body, 0)

    for b in range(NBUF):
        pltpu.make_async_copy(bufs[b], acc.at[rowb.at[0, b]], ssem[b]).wait()

    plsc.subcore_barrier()
    pltpu.sync_copy(acc.at[pl.ds(s * ROWS_PER_TILE, ROWS_PER_TILE)],
                    out_hbm.at[c, pl.ds(s * ROWS_PER_TILE, ROWS_PER_TILE)])

    @pl.when(s == 0)
    def _dump_tail():
        pltpu.sync_copy(acc.at[pl.ds(NS * ROWS_PER_TILE, ROWS_TAIL)],
                        out_hbm.at[c, pl.ds(NS * ROWS_PER_TILE, ROWS_TAIL)])


def _combine_body(p_ref, o_ref):
    o_ref[...] = jnp.maximum(p_ref[0] + p_ref[1], 0.0)


def _combine(partials):
    return pl.pallas_call(
        _combine_body,
        grid=(10,),
        in_specs=[pl.BlockSpec((NC, 1000, D), lambda i: (0, i, 0))],
        out_specs=pl.BlockSpec((1000, D), lambda i: (i, 0)),
        out_shape=jax.ShapeDtypeStruct((N, D), jnp.float32),
    )(partials)


def kernel(X0, edge_index, edge_weight, W, b):
    ei = edge_index.astype(jnp.int32)
    row2 = ei[0].reshape(NROWS_E, CHUNK)
    col2 = ei[1].reshape(NROWS_E, CHUNK)
    w2 = edge_weight.reshape(NROWS_E, CHUNK)
    h = _linear(X0, W, b.reshape(1, D))
    zeros = jnp.zeros((N, D), jnp.float32)
    partials = _spmm(h, col2, row2, w2, zeros)
    return _combine(partials)


# speedup vs baseline: 9.8525x; 1.0059x over previous
import functools

import jax
import jax.numpy as jnp
from jax import lax
from jax.experimental import pallas as pl
from jax.experimental.pallas import tpu as pltpu
from jax.experimental.pallas import tpu_sc as plsc

N = 10000
D = 128
E = 320000
NC = 2
NS = 16
NW = NC * NS
LANES = 16
CHUNK = 80
NROWS_E = E // CHUNK
CPB = 8
RSTRIDE = NW * CPB
FULL_ROUNDS = NROWS_E // RSTRIDE
EXTRA_TILES = (NROWS_E - FULL_ROUNDS * RSTRIDE) // CPB
NBUF = 4
ROWS_PER_TILE = 624
ROWS_TAIL = N - NS * ROWS_PER_TILE


def _linear_body(x_ref, w_ref, b_ref, o_ref):
    o_ref[...] = (
        lax.dot_general(x_ref[...], w_ref[...], (((1,), (1,)), ((), ())),
                        preferred_element_type=jnp.float32)
        + b_ref[...]
    )


def _linear(X0, W, b2):
    return pl.pallas_call(
        _linear_body,
        grid=(10,),
        in_specs=[
            pl.BlockSpec((1000, D), lambda i: (i, 0)),
            pl.BlockSpec((D, D), lambda i: (0, 0)),
            pl.BlockSpec((1, D), lambda i: (0, 0)),
        ],
        out_specs=pl.BlockSpec((1000, D), lambda i: (i, 0)),
        out_shape=jax.ShapeDtypeStruct((N, D), jnp.float32),
    )(X0, W, b2)


_MESH = plsc.VectorSubcoreMesh(core_axis_name="c", subcore_axis_name="s")


@functools.partial(
    pl.kernel,
    out_type=jax.ShapeDtypeStruct((NC, N, D), jnp.float32),
    mesh=_MESH,
    scratch_types=(
        [
            pltpu.VMEM((2, CPB, CHUNK), jnp.int32),
            pltpu.VMEM((2, CPB, CHUNK), jnp.int32),
            pltpu.VMEM((2, CPB, CHUNK), jnp.float32),
        ]
        + [pltpu.VMEM((CHUNK, D), jnp.float32) for _ in range(NBUF)]
        + [pltpu.VMEM_SHARED((N, D), jnp.float32)]
        + [pltpu.SemaphoreType.DMA for _ in range(2 * NBUF + 1)]
    ),
)
def _spmm(h_hbm, col_hbm, row_hbm, w_hbm, zeros_hbm, out_hbm, *scr):
    colb, rowb, wtb = scr[0], scr[1], scr[2]
    bufs = scr[3:3 + NBUF]
    acc = scr[3 + NBUF]
    gsem = scr[4 + NBUF:4 + 2 * NBUF]
    ssem = scr[4 + 2 * NBUF:4 + 3 * NBUF]
    isem = scr[4 + 3 * NBUF]

    c = lax.axis_index("c")
    s = lax.axis_index("s")
    wid = c * NS + s

    pltpu.sync_copy(zeros_hbm.at[pl.ds(s * ROWS_PER_TILE, ROWS_PER_TILE)],
                    acc.at[pl.ds(s * ROWS_PER_TILE, ROWS_PER_TILE)])

    @pl.when(s == 0)
    def _init_tail():
        pltpu.sync_copy(zeros_hbm.at[pl.ds(NS * ROWS_PER_TILE, ROWS_TAIL)],
                        acc.at[pl.ds(NS * ROWS_PER_TILE, ROWS_TAIL)])

    plsc.subcore_barrier()

    br0 = wid * CPB
    pltpu.sync_copy(col_hbm.at[pl.ds(br0, CPB)], colb.at[0])
    pltpu.sync_copy(row_hbm.at[pl.ds(br0, CPB)], rowb.at[0])
    pltpu.sync_copy(w_hbm.at[pl.ds(br0, CPB)], wtb.at[0])

    nblk = FULL_ROUNDS + jnp.where(wid < EXTRA_TILES, 1, 0)

    def block_body(r, carry):
        slot = lax.rem(r, 2)
        nslot = 1 - slot
        nbr = (r + 1) * RSTRIDE + wid * CPB

        @pl.when(r + 1 < nblk)
        def _prefetch():
            pltpu.async_copy(col_hbm.at[pl.ds(nbr, CPB)], colb.at[nslot], isem)
            pltpu.async_copy(row_hbm.at[pl.ds(nbr, CPB)], rowb.at[nslot], isem)
            pltpu.async_copy(w_hbm.at[pl.ds(nbr, CPB)], wtb.at[nslot], isem)

        def _reuse_gather(p, first_block_cond):
            b = p % NBUF

            @pl.when(first_block_cond)
            def _drain():
                pltpu.make_async_copy(bufs[b], acc.at[rowb.at[slot, p]],
                                      ssem[b]).wait()

            pltpu.async_copy(h_hbm.at[colb.at[slot, p]], bufs[b], gsem[b])

        _reuse_gather(0, r > 0)
        _reuse_gather(1, r > 0)

        for k in range(CPB):
            b = k % NBUF
            p = k + 2
            if p < CPB:
                _reuse_gather(p, (r > 0) if p < NBUF else (r >= 0))
            pltpu.make_async_copy(h_hbm.at[colb.at[slot, k]], bufs[b],
                                  gsem[b]).wait()
            def group_body(g, carry2, b=b, k=k):
                e0 = g * LANES
                w16 = wtb[slot, k, pl.ds(e0, LANES)]
                for rr in range(LANES):
                    wbc = w16[jnp.full((LANES,), rr, jnp.int32)]
                    e = e0 + rr
                    for j in range(D // LANES):
                        sl = bufs[b][e, pl.ds(j * LANES, LANES)]
                        bufs[b][e, pl.ds(j * LANES, LANES)] = sl * wbc
                return carry2

            lax.fori_loop(0, CHUNK // LANES, group_body, 0)

            pltpu.async_copy(bufs[b], acc.at[rowb.at[slot, k]], ssem[b],
                             add=True)

        @pl.when(r + 1 < nblk)
        def _wait_prefetch():
            pltpu.make_async_copy(col_hbm.at[pl.ds(nbr, CPB)], colb.at[nslot],
                                  isem).wait()
            pltpu.make_async_copy(row_hbm.at[pl.ds(nbr, CPB)], rowb.at[nslot],
                                  isem).wait()
            pltpu.make_async_copy(w_hbm.at[pl.ds(nbr, CPB)], wtb.at[nslot],
                                  isem).wait()
        return carry

    lax.fori_loop(0, nblk, block_body, 0)

    for b in range(NBUF):
        pltpu.make_async_copy(bufs[b], acc.at[rowb.at[0, b]], ssem[b]).wait()

    plsc.subcore_barrier()
    pltpu.sync_copy(acc.at[pl.ds(s * ROWS_PER_TILE, ROWS_PER_TILE)],
                    out_hbm.at[c, pl.ds(s * ROWS_PER_TILE, ROWS_PER_TILE)])

    @pl.when(s == 0)
    def _dump_tail():
        pltpu.sync_copy(acc.at[pl.ds(NS * ROWS_PER_TILE, ROWS_TAIL)],
                        out_hbm.at[c, pl.ds(NS * ROWS_PER_TILE, ROWS_TAIL)])


def _combine_body(p_ref, o_ref):
    o_ref[...] = jnp.maximum(p_ref[0] + p_ref[1], 0.0)


def _combine(partials):
    return pl.pallas_call(
        _combine_body,
        grid=(10,),
        in_specs=[pl.BlockSpec((NC, 1000, D), lambda i: (0, i, 0))],
        out_specs=pl.BlockSpec((1000, D), lambda i: (i, 0)),
        out_shape=jax.ShapeDtypeStruct((N, D), jnp.float32),
    )(partials)


def kernel(X0, edge_index, edge_weight, W, b):
    ei = edge_index.astype(jnp.int32)
    row2 = ei[0].reshape(NROWS_E, CHUNK)
    col2 = ei[1].reshape(NROWS_E, CHUNK)
    w2 = edge_weight.reshape(NROWS_E, CHUNK)
    h = _linear(X0, W, b.reshape(1, D))
    zeros = jnp.zeros((N, D), jnp.float32)
    partials = _spmm(h, col2, row2, w2, zeros)
    return _combine(partials)
